# Initial kernel scaffold; baseline (speedup 1.0000x reference)
import jax
import jax.numpy as jnp
from jax.experimental import pallas as pl


def kernel(x, edge_index, eps1, W1, b1, eps2, W2, b2, eps3, W3, b3):
    raise NotImplementedError("write your pallas kernel here")


# speedup vs baseline: 4.3231x; 4.3231x over previous
import functools

import jax
import jax.numpy as jnp
from jax import lax
from jax.experimental import pallas as pl
from jax.experimental.pallas import tpu as pltpu
from jax.experimental.pallas import tpu_sc as plsc

N_NODES = 10000
N_EDGES = 320000
D = 128

NC = 2
NS = 16
NW = NC * NS
L = 16

CHUNK = 128
CPW = 79
E_PAD = NW * CPW * CHUNK
DUMP_ROW = N_NODES
AGG_ROWS = 10240
ROWS_PER_TILE = AGG_ROWS // NS


def _make_sc_agg():
    mesh = plsc.VectorSubcoreMesh(core_axis_name="c", subcore_axis_name="s")

    @functools.partial(
        pl.kernel,
        mesh=mesh,
        out_type=jax.ShapeDtypeStruct((NC, AGG_ROWS, D), jnp.float32),
        scratch_types=[
            pltpu.VMEM((CPW, CHUNK), jnp.int32),
            pltpu.VMEM((CPW, CHUNK), jnp.int32),
            pltpu.VMEM((CHUNK, D), jnp.float32),
            pltpu.VMEM_SHARED((AGG_ROWS, D), jnp.float32),
            pltpu.SemaphoreType.DMA,
        ],
    )
    def sc_agg(h_hbm, src_hbm, dst_hbm, out_hbm, src_v, dst_v, rows_v, agg_sh, sem):
        c = lax.axis_index("c")
        s = lax.axis_index("s")
        wid = c * NS + s

        zvec = jnp.zeros((L,), jnp.float32)

        def zero_body(r, carry):
            for k in range(D // L):
                rows_v[r, pl.ds(k * L, L)] = zvec
            return carry

        lax.fori_loop(0, CHUNK, zero_body, 0)
        base = s * ROWS_PER_TILE
        for t in range(ROWS_PER_TILE // CHUNK):
            pltpu.sync_copy(rows_v, agg_sh.at[pl.ds(base + t * CHUNK, CHUNK)])
        plsc.subcore_barrier()

        pltpu.sync_copy(src_hbm.at[wid], src_v)
        pltpu.sync_copy(dst_hbm.at[wid], dst_v)

        def chunk_body(j, carry):
            pltpu.async_copy(h_hbm.at[src_v.at[j]], rows_v, sem).wait()
            pltpu.sync_copy(rows_v, agg_sh.at[dst_v.at[j]], add=True)
            return carry

        lax.fori_loop(0, CPW, chunk_body, 0)
        plsc.subcore_barrier()

        pltpu.sync_copy(agg_sh.at[pl.ds(base, ROWS_PER_TILE)],
                        out_hbm.at[c, pl.ds(base, ROWS_PER_TILE)])

    return sc_agg


def _tc_layer(h, agg, W, b, eps, act):
    BLK = 1000

    def body(eps_ref, h_ref, agg_ref, W_ref, b_ref, o_ref):
        y = h_ref[...] * (1.0 + eps_ref[0]) + agg_ref[0] + agg_ref[1]
        z = jnp.dot(y, W_ref[...], preferred_element_type=jnp.float32) + b_ref[...]
        if act == "relu":
            z = jnp.maximum(z, 0.0)
        elif act == "log_softmax":
            m = jnp.max(z, axis=-1, keepdims=True)
            z = z - m
            z = z - jnp.log(jnp.sum(jnp.exp(z), axis=-1, keepdims=True))
        o_ref[...] = z

    return pl.pallas_call(
        body,
        grid=(N_NODES // BLK,),
        in_specs=[
            pl.BlockSpec(memory_space=pltpu.SMEM),
            pl.BlockSpec((BLK, D), lambda i: (i, 0)),
            pl.BlockSpec((2, BLK, D), lambda i: (0, i, 0)),
            pl.BlockSpec((D, D), lambda i: (0, 0)),
            pl.BlockSpec((1, D), lambda i: (0, 0)),
        ],
        out_specs=pl.BlockSpec((BLK, D), lambda i: (i, 0)),
        out_shape=jax.ShapeDtypeStruct((N_NODES, D), jnp.float32),
    )(jnp.reshape(eps, (1,)), h, agg, W, jnp.reshape(b, (1, D)))


def kernel(x, edge_index, eps1, W1, b1, eps2, W2, b2, eps3, W3, b3):
    src = edge_index[0].astype(jnp.int32)
    dst = edge_index[1].astype(jnp.int32)
    pad = E_PAD - N_EDGES
    srcp = jnp.concatenate([src, jnp.zeros((pad,), jnp.int32)]).reshape(NW, CPW, CHUNK)
    dstp = jnp.concatenate([dst, jnp.full((pad,), DUMP_ROW, jnp.int32)]).reshape(NW, CPW, CHUNK)

    sc_agg = _make_sc_agg()

    h = x
    for eps, W, b, act in ((eps1, W1, b1, "relu"),
                           (eps2, W2, b2, "relu"),
                           (eps3, W3, b3, "log_softmax")):
        agg = sc_agg(h, srcp, dstp)
        h = _tc_layer(h, agg, W, b, eps, act)
    return h
